# baseline (speedup 1.0000x reference)
import functools
import jax
import jax.numpy as jnp
from jax import lax
from jax.experimental import pallas as pl
from jax.experimental.pallas import tpu as pltpu
from jax.experimental.pallas import tpu_sc as plsc

ARITY = 8
G = 256
P = 256
D = 16

NC = 2
NS = 16
NW = NC * NS

R = 8
XCH = R * G * ARITY
NIDX = R * G
NIVEC = NIDX // 16
IDX_PER_DMA = 2048
NDMA = NIDX // IDX_PER_DMA


def _sc_body(x_hbm, p_hbm, o_hbm,
             x_a, x_b, idx_a, idx_b, min_a, min_b, gath_a, gath_b,
             semx_a, semx_b, semg_a, semg_b):
    wid = lax.axis_index("s") * NC + lax.axis_index("c")
    iota = lax.iota(jnp.int32, 16)
    iota8 = iota * 8

    nchunk = (o_hbm.shape[0] // G) // (NW * R)
    base_row = wid * nchunk * R

    def start_x(c, x_buf, semx):
        pltpu.async_copy(
            x_hbm.at[pl.ds((base_row + c * R) * G * ARITY, XCH)],
            x_buf, semx)

    def s1(x_buf, idx_buf, min_buf, gath_buf, semx, semg):
        pltpu.make_async_copy(x_hbm.at[pl.ds(0, XCH)], x_buf, semx).wait()

        @plsc.parallel_loop(0, NIVEC, unroll=4)
        def _ivec(i):
            base = i * 128 + iota8
            x0 = plsc.load_gather(x_buf, [base])
            m = jnp.abs(x0)
            ind = (x0 >= 0).astype(jnp.int32)
            for j in range(1, ARITY):
                xj = plsc.load_gather(x_buf, [base + j])
                m = jnp.minimum(m, jnp.abs(xj))
                ind = jnp.bitwise_or(
                    ind, jnp.left_shift((xj >= 0).astype(jnp.int32), j))
            gg = i * 16 + iota
            prow = jnp.bitwise_or(
                jnp.left_shift(jnp.bitwise_and(gg, G - 1), 8), ind)
            idx_buf[pl.ds(i * 16, 16)] = prow
            min_buf[pl.ds(i * 16, 16)] = m

        for j in range(NDMA):
            pltpu.async_copy(
                p_hbm.at[idx_buf.at[pl.ds(j * IDX_PER_DMA, IDX_PER_DMA)]],
                gath_buf.at[pl.ds(j * IDX_PER_DMA, IDX_PER_DMA)],
                semg)

    def s2(c, idx_buf, min_buf, gath_buf, semg):
        for j in range(NDMA):
            pltpu.make_async_copy(
                p_hbm.at[idx_buf.at[pl.ds(j * IDX_PER_DMA, IDX_PER_DMA)]],
                gath_buf.at[pl.ds(j * IDX_PER_DMA, IDX_PER_DMA)],
                semg).wait()

        @plsc.parallel_loop(0, NIDX // 16, unroll=4)
        def _scale(b):
            minvec = min_buf[pl.ds(b * 16, 16)]
            rbase = b * 16 + iota
            for d in range(D):
                dvec = jnp.full((16,), d, jnp.int32)
                vals = plsc.load_gather(gath_buf, [rbase, dvec])
                plsc.store_scatter(gath_buf, [rbase, dvec], vals * minvec)

        pltpu.sync_copy(gath_buf,
                        o_hbm.at[pl.ds((base_row + c * R) * G, NIDX)])

    start_x(0, x_a, semx_a)

    @pl.loop(0, nchunk // 2)
    def _pair(k):
        c0 = 2 * k
        s1(x_a, idx_a, min_a, gath_a, semx_a, semg_a)
        start_x(c0 + 1, x_b, semx_b)
        s2(c0, idx_a, min_a, gath_a, semg_a)
        s1(x_b, idx_b, min_b, gath_b, semx_b, semg_b)

        @pl.when(c0 + 2 < nchunk)
        def _():
            start_x(c0 + 2, x_a, semx_a)

        s2(c0 + 1, idx_b, min_b, gath_b, semg_b)


@jax.jit
def kernel(X, params):
    B = X.shape[0]
    Xf = X.reshape(B * G * ARITY)
    Pf = params.reshape(G * P, D)
    mesh = plsc.VectorSubcoreMesh(core_axis_name="c", subcore_axis_name="s")
    run = functools.partial(
        pl.kernel,
        out_type=jax.ShapeDtypeStruct((B * G, D), jnp.float32),
        mesh=mesh,
        compiler_params=pltpu.CompilerParams(
            needs_layout_passes=False, use_tc_tiling_on_sc=False),
        scratch_types=[
            pltpu.VMEM((XCH,), jnp.float32),
            pltpu.VMEM((XCH,), jnp.float32),
            pltpu.VMEM((NIDX,), jnp.int32),
            pltpu.VMEM((NIDX,), jnp.int32),
            pltpu.VMEM((NIDX,), jnp.float32),
            pltpu.VMEM((NIDX,), jnp.float32),
            pltpu.VMEM((NIDX, D), jnp.float32),
            pltpu.VMEM((NIDX, D), jnp.float32),
            pltpu.SemaphoreType.DMA,
            pltpu.SemaphoreType.DMA,
            pltpu.SemaphoreType.DMA,
            pltpu.SemaphoreType.DMA,
        ],
    )(_sc_body)
    out = run(Xf, Pf)
    return out.reshape(B, G * D)


# speedup vs baseline: 1.2837x; 1.2837x over previous
import functools
import jax
import jax.numpy as jnp
from jax import lax
from jax.experimental import pallas as pl
from jax.experimental.pallas import tpu as pltpu
from jax.experimental.pallas import tpu_sc as plsc

ARITY = 8
G = 256
P = 256
D = 16

NC = 2
NS = 16
NW = NC * NS

R = 8
XCH = R * G * ARITY
NIDX = R * G
NIVEC = NIDX // 16
IDX_PER_DMA = 2048
NDMA = NIDX // IDX_PER_DMA


def _sc_body(x_hbm, p_hbm, o_hbm,
             x_a, x_b, idx_a, idx_b, min_a, min_b, gath_a, gath_b,
             semx_a, semx_b, semg_a, semg_b):
    wid = lax.axis_index("s") * NC + lax.axis_index("c")
    iota = lax.iota(jnp.int32, 16)
    iota8 = iota * 8

    nchunk = (o_hbm.shape[0] // G) // (NW * R)
    base_row = wid * nchunk * R

    def start_x(c, x_buf, semx):
        pltpu.async_copy(
            x_hbm.at[pl.ds((base_row + c * R) * G * ARITY, XCH)],
            x_buf, semx)

    def s1(x_buf, idx_buf, min_buf, gath_buf, semx, semg):
        pltpu.make_async_copy(x_hbm.at[pl.ds(0, XCH)], x_buf, semx).wait()

        @plsc.parallel_loop(0, NIVEC, unroll=4)
        def _ivec(i):
            base = i * 128 + iota8
            x0 = plsc.load_gather(x_buf, [base])
            m = jnp.abs(x0)
            ind = (x0 >= 0).astype(jnp.int32)
            for j in range(1, ARITY):
                xj = plsc.load_gather(x_buf, [base + j])
                m = jnp.minimum(m, jnp.abs(xj))
                ind = jnp.bitwise_or(
                    ind, jnp.left_shift((xj >= 0).astype(jnp.int32), j))
            gg = i * 16 + iota
            prow = jnp.bitwise_or(
                jnp.left_shift(jnp.bitwise_and(gg, G - 1), 8), ind)
            idx_buf[pl.ds(i * 16, 16)] = prow
            min_buf[pl.ds(i * 16, 16)] = m

        for j in range(NDMA):
            pltpu.async_copy(
                p_hbm.at[idx_buf.at[pl.ds(j * IDX_PER_DMA, IDX_PER_DMA)]],
                gath_buf.at[pl.ds(j * IDX_PER_DMA, IDX_PER_DMA)],
                semg)

    def s2(c, idx_buf, min_buf, gath_buf, semg):
        for j in range(NDMA):
            pltpu.make_async_copy(
                p_hbm.at[idx_buf.at[pl.ds(j * IDX_PER_DMA, IDX_PER_DMA)]],
                gath_buf.at[pl.ds(j * IDX_PER_DMA, IDX_PER_DMA)],
                semg).wait()

        @plsc.parallel_loop(0, NIDX // 16, unroll=1)
        def _scale(b):
            minvec = min_buf[pl.ds(b * 16, 16)]
            rbase = b * 16 + iota
            for d in range(D):
                dvec = jnp.full((16,), d, jnp.int32)
                vals = plsc.load_gather(gath_buf, [rbase, dvec])
                plsc.store_scatter(gath_buf, [rbase, dvec], vals * minvec)

        pltpu.sync_copy(gath_buf,
                        o_hbm.at[pl.ds((base_row + c * R) * G, NIDX)])

    start_x(0, x_a, semx_a)

    @pl.loop(0, nchunk // 2)
    def _pair(k):
        c0 = 2 * k
        s1(x_a, idx_a, min_a, gath_a, semx_a, semg_a)
        start_x(c0 + 1, x_b, semx_b)
        s2(c0, idx_a, min_a, gath_a, semg_a)
        s1(x_b, idx_b, min_b, gath_b, semx_b, semg_b)

        @pl.when(c0 + 2 < nchunk)
        def _():
            start_x(c0 + 2, x_a, semx_a)

        s2(c0 + 1, idx_b, min_b, gath_b, semg_b)


@jax.jit
def kernel(X, params):
    B = X.shape[0]
    Xf = X.reshape(B * G * ARITY)
    Pf = params.reshape(G * P, D)
    mesh = plsc.VectorSubcoreMesh(core_axis_name="c", subcore_axis_name="s")
    run = functools.partial(
        pl.kernel,
        out_type=jax.ShapeDtypeStruct((B * G, D), jnp.float32),
        mesh=mesh,
        compiler_params=pltpu.CompilerParams(
            needs_layout_passes=False, use_tc_tiling_on_sc=False),
        scratch_types=[
            pltpu.VMEM((XCH,), jnp.float32),
            pltpu.VMEM((XCH,), jnp.float32),
            pltpu.VMEM((NIDX,), jnp.int32),
            pltpu.VMEM((NIDX,), jnp.int32),
            pltpu.VMEM((NIDX,), jnp.float32),
            pltpu.VMEM((NIDX,), jnp.float32),
            pltpu.VMEM((NIDX, D), jnp.float32),
            pltpu.VMEM((NIDX, D), jnp.float32),
            pltpu.SemaphoreType.DMA,
            pltpu.SemaphoreType.DMA,
            pltpu.SemaphoreType.DMA,
            pltpu.SemaphoreType.DMA,
        ],
    )(_sc_body)
    out = run(Xf, Pf)
    return out.reshape(B, G * D)


# speedup vs baseline: 1.3041x; 1.0159x over previous
import functools
import jax
import jax.numpy as jnp
from jax import lax
from jax.experimental import pallas as pl
from jax.experimental.pallas import tpu as pltpu
from jax.experimental.pallas import tpu_sc as plsc

ARITY = 8
G = 256
P = 256
D = 16

NC = 2
NS = 16
NW = NC * NS

R = 8
XCH = R * G * ARITY
NIDX = R * G
NIVEC = NIDX // 16
IDX_PER_DMA = 2048
NDMA = NIDX // IDX_PER_DMA


def _sc_body(x_hbm, p_hbm, o_hbm,
             x_a, x_b, idx_a, idx_b, min_a, min_b, gath_a, gath_b,
             semx_a, semx_b, semg_a, semg_b):
    wid = lax.axis_index("s") * NC + lax.axis_index("c")
    iota = lax.iota(jnp.int32, 16)
    iota8 = iota * 8

    nchunk = (o_hbm.shape[0] // G) // (NW * R)
    base_row = wid * nchunk * R

    def start_x(c, x_buf, semx):
        pltpu.async_copy(
            x_hbm.at[pl.ds((base_row + c * R) * G * ARITY, XCH)],
            x_buf, semx)

    def s1(x_buf, idx_buf, min_buf, gath_buf, semx, semg):
        pltpu.make_async_copy(x_hbm.at[pl.ds(0, XCH)], x_buf, semx).wait()

        @plsc.parallel_loop(0, NIVEC, unroll=1)
        def _ivec(i):
            base = i * 128 + iota8
            x0 = plsc.load_gather(x_buf, [base])
            m = jnp.abs(x0)
            ind = (x0 >= 0).astype(jnp.int32)
            for j in range(1, ARITY):
                xj = plsc.load_gather(x_buf, [base + j])
                m = jnp.minimum(m, jnp.abs(xj))
                ind = jnp.bitwise_or(
                    ind, jnp.left_shift((xj >= 0).astype(jnp.int32), j))
            gg = i * 16 + iota
            prow = jnp.bitwise_or(
                jnp.left_shift(jnp.bitwise_and(gg, G - 1), 8), ind)
            idx_buf[pl.ds(i * 16, 16)] = prow
            min_buf[pl.ds(i * 16, 16)] = m

        for j in range(NDMA):
            pltpu.async_copy(
                p_hbm.at[idx_buf.at[pl.ds(j * IDX_PER_DMA, IDX_PER_DMA)]],
                gath_buf.at[pl.ds(j * IDX_PER_DMA, IDX_PER_DMA)],
                semg)

    def s2(c, idx_buf, min_buf, gath_buf, semg):
        for j in range(NDMA):
            pltpu.make_async_copy(
                p_hbm.at[idx_buf.at[pl.ds(j * IDX_PER_DMA, IDX_PER_DMA)]],
                gath_buf.at[pl.ds(j * IDX_PER_DMA, IDX_PER_DMA)],
                semg).wait()

        @plsc.parallel_loop(0, NIDX // 16, unroll=1)
        def _scale(b):
            minvec = min_buf[pl.ds(b * 16, 16)]
            rbase = b * 16 + iota
            for d in range(D):
                dvec = jnp.full((16,), d, jnp.int32)
                vals = plsc.load_gather(gath_buf, [rbase, dvec])
                plsc.store_scatter(gath_buf, [rbase, dvec], vals * minvec)

        pltpu.sync_copy(gath_buf,
                        o_hbm.at[pl.ds((base_row + c * R) * G, NIDX)])

    start_x(0, x_a, semx_a)

    @pl.loop(0, nchunk // 2)
    def _pair(k):
        c0 = 2 * k
        s1(x_a, idx_a, min_a, gath_a, semx_a, semg_a)
        start_x(c0 + 1, x_b, semx_b)
        s2(c0, idx_a, min_a, gath_a, semg_a)
        s1(x_b, idx_b, min_b, gath_b, semx_b, semg_b)

        @pl.when(c0 + 2 < nchunk)
        def _():
            start_x(c0 + 2, x_a, semx_a)

        s2(c0 + 1, idx_b, min_b, gath_b, semg_b)


@jax.jit
def kernel(X, params):
    B = X.shape[0]
    Xf = X.reshape(B * G * ARITY)
    Pf = params.reshape(G * P, D)
    mesh = plsc.VectorSubcoreMesh(core_axis_name="c", subcore_axis_name="s")
    run = functools.partial(
        pl.kernel,
        out_type=jax.ShapeDtypeStruct((B * G, D), jnp.float32),
        mesh=mesh,
        compiler_params=pltpu.CompilerParams(
            needs_layout_passes=False, use_tc_tiling_on_sc=False),
        scratch_types=[
            pltpu.VMEM((XCH,), jnp.float32),
            pltpu.VMEM((XCH,), jnp.float32),
            pltpu.VMEM((NIDX,), jnp.int32),
            pltpu.VMEM((NIDX,), jnp.int32),
            pltpu.VMEM((NIDX,), jnp.float32),
            pltpu.VMEM((NIDX,), jnp.float32),
            pltpu.VMEM((NIDX, D), jnp.float32),
            pltpu.VMEM((NIDX, D), jnp.float32),
            pltpu.SemaphoreType.DMA,
            pltpu.SemaphoreType.DMA,
            pltpu.SemaphoreType.DMA,
            pltpu.SemaphoreType.DMA,
        ],
    )(_sc_body)
    out = run(Xf, Pf)
    return out.reshape(B, G * D)
